# baseline (speedup 1.0000x reference)
import jax
import jax.numpy as jnp
from jax import lax
from jax.experimental import pallas as pl
from jax.experimental.pallas import tpu as pltpu
from jax.experimental.pallas import tpu_sc as plsc

D = 128
NN = 10000
NE = 320000
NREL = 38
NRELP = 40
NP = 10240
NC, NS = 2, 16
NW = NC * NS
CH = 80
NE3 = 327680
EPT = NE3 // NW
NCH = EPT // CH
APT = NP // NW

_mesh = plsc.VectorSubcoreMesh(
    core_axis_name="c", subcore_axis_name="s", num_cores=NC, num_subcores=NS)
_sc_params = pltpu.CompilerParams(needs_layout_passes=False)


def _wid():
    return lax.axis_index("s") * NC + lax.axis_index("c")


def _gather_x_body(ce_hbm, cid_hbm, x_hbm, idx_v, rows_v, sem):
    base = _wid() * APT

    def step(k, carry):
        off = base + k * CH
        pltpu.sync_copy(cid_hbm.at[pl.ds(off, CH)], idx_v)
        pltpu.async_copy(ce_hbm.at[idx_v], rows_v, sem).wait()
        pltpu.sync_copy(rows_v, x_hbm.at[pl.ds(off, CH)])
        return carry

    lax.fori_loop(0, APT // CH, step, 0)


_gather_x = pl.kernel(
    _gather_x_body,
    out_type=jax.ShapeDtypeStruct((NP, D), jnp.float32),
    mesh=_mesh,
    compiler_params=_sc_params,
    scratch_types=[
        pltpu.VMEM((CH,), jnp.int32),
        pltpu.VMEM((CH, D), jnp.float32),
        pltpu.SemaphoreType.DMA,
    ],
)


def _node_pre_body(x_ref, wm_ref, bm_ref, ws_ref, rp_ref, wl_ref,
                   xm_ref, xs_ref, relcat_ref):
    x = x_ref[...]
    xm_ref[...] = jnp.dot(x, wm_ref[0:D, :],
                          preferred_element_type=jnp.float32) + bm_ref[...]
    xs_ref[...] = jnp.dot(x, ws_ref[...], preferred_element_type=jnp.float32)
    rp = rp_ref[...]
    rm = jnp.dot(rp, wm_ref[D:2 * D, :], preferred_element_type=jnp.float32)
    rl = jnp.dot(rp, wl_ref[D:2 * D, :], preferred_element_type=jnp.float32)
    relcat_ref[...] = jnp.concatenate([rm, rl], axis=1)


_node_pre = pl.pallas_call(
    _node_pre_body,
    out_shape=[
        jax.ShapeDtypeStruct((NP, D), jnp.float32),
        jax.ShapeDtypeStruct((NP, D), jnp.float32),
        jax.ShapeDtypeStruct((NRELP, 2 * D), jnp.float32),
    ],
)


EB = 2048


def _edge_bias_body(attr_ref, relcat_ref, wrm_ref, wrl_ref):
    attr = attr_ref[...]
    reli = attr[:, 0:1].astype(jnp.int32)
    w = attr[:, 1:2]
    io = lax.broadcasted_iota(jnp.int32, (EB, NRELP), 1)
    ohw = jnp.where(reli == io, w, 0.0)
    big = jnp.dot(ohw, relcat_ref[...], preferred_element_type=jnp.float32)
    wrm_ref[...] = big[:, 0:D]
    wrl_ref[...] = big[:, D:2 * D]


_edge_bias = pl.pallas_call(
    _edge_bias_body,
    grid=(NE3 // EB,),
    in_specs=[
        pl.BlockSpec((EB, 2), lambda i: (i, 0)),
        pl.BlockSpec((NRELP, 2 * D), lambda i: (0, 0)),
    ],
    out_specs=[
        pl.BlockSpec((EB, D), lambda i: (i, 0)),
        pl.BlockSpec((EB, D), lambda i: (i, 0)),
    ],
    out_shape=[
        jax.ShapeDtypeStruct((NE3, D), jnp.float32),
        jax.ShapeDtypeStruct((NE3, D), jnp.float32),
    ],
)


def _msg_agg_body(xm_hbm, wrm_hbm, src_hbm, dst_hbm, agg_hbm,
                  is0, is1, id0, id1, r0, r1, w0, w1, shared,
                  sg0, sg1, sa0, sa1, si0, si1, sd0, sd1):
    c = lax.axis_index("c")
    s = lax.axis_index("s")
    wid = s * NC + c
    nstripe = NP // NS
    iss = (is0, is1)
    ids = (id0, id1)
    rows = (r0, r1)
    wrms = (w0, w1)
    sgs = (sg0, sg1)
    sas = (sa0, sa1)
    sis = (si0, si1)
    sds = (sd0, sd1)

    zero = jnp.zeros((16,), jnp.float32)
    for r in range(40):
        for j in range(8):
            r0[r, pl.ds(16 * j, 16)] = zero

    def zstep(i, carry):
        pltpu.sync_copy(r0.at[pl.ds(0, 40)],
                        shared.at[pl.ds(s * nstripe + i * 40, 40)])
        return carry

    lax.fori_loop(0, nstripe // 40, zstep, 0)
    plsc.subcore_barrier()

    def issue_is(k, b):
        pltpu.async_copy(src_hbm.at[pl.ds(wid * EPT + k * CH, CH)],
                         iss[b], sis[b])

    def wait_is(k, b):
        pltpu.make_async_copy(src_hbm.at[pl.ds(wid * EPT + k * CH, CH)],
                              iss[b], sis[b]).wait()

    def issue_id(k, b):
        pltpu.async_copy(dst_hbm.at[pl.ds(wid * EPT + k * CH, CH)],
                         ids[b], sds[b])

    def wait_id(k, b):
        pltpu.make_async_copy(dst_hbm.at[pl.ds(wid * EPT + k * CH, CH)],
                              ids[b], sds[b]).wait()

    def issue_g(k, b):
        pltpu.async_copy(xm_hbm.at[iss[b]], rows[b], sgs[b])
        pltpu.async_copy(wrm_hbm.at[pl.ds(wid * EPT + k * CH, CH)],
                         wrms[b], sgs[b])

    def wait_g(k, b):
        pltpu.make_async_copy(xm_hbm.at[iss[b]], rows[b], sgs[b]).wait()
        pltpu.make_async_copy(wrm_hbm.at[pl.ds(wid * EPT + k * CH, CH)],
                              wrms[b], sgs[b]).wait()

    def sadd(k, b):
        pltpu.async_copy(rows[b], shared.at[ids[b]], sas[b], add=True)

    def wait_sa(k, b):
        pltpu.make_async_copy(rows[b], shared.at[ids[b]], sas[b]).wait()

    def compute(b):
        rv, wv = rows[b], wrms[b]

        def rstep(r, c2):
            for j in range(8):
                sl = pl.ds(16 * j, 16)
                rv[r, sl] = jnp.maximum(rv[r, sl] + wv[r, sl], 0.0)
            return c2

        lax.fori_loop(0, CH, rstep, 0)

    pltpu.sync_copy(src_hbm.at[pl.ds(wid * EPT, CH)], is0)
    pltpu.sync_copy(dst_hbm.at[pl.ds(wid * EPT, CH)], id0)
    pltpu.sync_copy(src_hbm.at[pl.ds(wid * EPT + CH, CH)], is1)
    issue_g(0, 0)

    def gstep(g, carry):
        k0 = 2 * g
        k1 = k0 + 1

        wait_g(k0, 0)
        compute(0)

        @pl.when(k0 >= 1)
        def _():
            wait_id(k0, 0)

        sadd(k0, 0)

        @pl.when(k0 >= 1)
        def _():
            wait_sa(k0 - 1, 1)
            wait_is(k1, 1)

        issue_g(k1, 1)
        issue_id(k1, 1)

        @pl.when(k0 + 2 < NCH)
        def _():
            issue_is(k0 + 2, 0)

        wait_g(k1, 1)
        compute(1)
        wait_id(k1, 1)
        sadd(k1, 1)

        @pl.when(k1 + 1 < NCH)
        def _():
            wait_sa(k0, 0)
            wait_is(k1 + 1, 0)
            issue_g(k1 + 1, 0)
            issue_id(k1 + 1, 0)

            @pl.when(k1 + 2 < NCH)
            def _():
                issue_is(k1 + 2, 1)

        return carry

    lax.fori_loop(0, NCH // 2, gstep, 0)
    wait_sa(NCH - 2, 0)
    wait_sa(NCH - 1, 1)
    plsc.subcore_barrier()
    pltpu.sync_copy(shared.at[pl.ds(s * nstripe, nstripe)],
                    agg_hbm.at[c, pl.ds(s * nstripe, nstripe)])


_msg_agg = pl.kernel(
    _msg_agg_body,
    out_type=jax.ShapeDtypeStruct((NC, NP, D), jnp.float32),
    mesh=_mesh,
    compiler_params=_sc_params,
    scratch_types=[
        pltpu.VMEM((CH,), jnp.int32),
        pltpu.VMEM((CH,), jnp.int32),
        pltpu.VMEM((CH,), jnp.int32),
        pltpu.VMEM((CH,), jnp.int32),
        pltpu.VMEM((CH, D), jnp.float32),
        pltpu.VMEM((CH, D), jnp.float32),
        pltpu.VMEM((CH, D), jnp.float32),
        pltpu.VMEM((CH, D), jnp.float32),
        pltpu.VMEM_SHARED((NP, D), jnp.float32),
        pltpu.SemaphoreType.DMA,
        pltpu.SemaphoreType.DMA,
        pltpu.SemaphoreType.DMA,
        pltpu.SemaphoreType.DMA,
        pltpu.SemaphoreType.DMA,
        pltpu.SemaphoreType.DMA,
        pltpu.SemaphoreType.DMA,
        pltpu.SemaphoreType.DMA,
    ],
)


def _node_upd_body(agg_ref, xs_ref, wu_ref, bu_ref, wl_ref, bl_ref,
                   ls_ref, ld_ref):
    aggs = agg_ref[0] + agg_ref[1]
    x2 = jnp.maximum(
        jnp.dot(aggs, wu_ref[...], preferred_element_type=jnp.float32)
        + xs_ref[...] + bu_ref[...], 0.0)
    ls_ref[...] = jnp.dot(x2, wl_ref[0:D, :],
                          preferred_element_type=jnp.float32) + bl_ref[...]
    ld_ref[...] = jnp.dot(x2, wl_ref[2 * D:3 * D, :],
                          preferred_element_type=jnp.float32)


_node_upd = pl.pallas_call(
    _node_upd_body,
    out_shape=[
        jax.ShapeDtypeStruct((NP, D), jnp.float32),
        jax.ShapeDtypeStruct((NP, D), jnp.float32),
    ],
)


def _edge_out_body(ls_hbm, ld_hbm, wrl_hbm, src_hbm, dst_hbm,
                   cid_hbm, enc_hbm, t0_hbm, t2_hbm,
                   isa, ida, cid_v,
                   a0, a1, b0, b1, c0, c1, u0, u1, v0, v1,
                   sg0, sg1, so0, so1):
    wid = _wid()
    avs = (a0, a1)
    bvs = (b0, b1)
    cvs = (c0, c1)
    uvs = (u0, u1)
    vvs = (v0, v1)
    sgs = (sg0, sg1)
    sos = (so0, so1)

    pltpu.sync_copy(src_hbm.at[pl.ds(wid * EPT, EPT)], isa)
    pltpu.sync_copy(dst_hbm.at[pl.ds(wid * EPT, EPT)], ida)
    pltpu.sync_copy(cid_hbm, cid_v)

    def issue(k, b):
        sl = pl.ds(k * CH, CH)
        pltpu.async_copy(ls_hbm.at[isa.at[sl]], avs[b], sgs[b])
        pltpu.async_copy(ld_hbm.at[ida.at[sl]], bvs[b], sgs[b])
        pltpu.async_copy(wrl_hbm.at[pl.ds(wid * EPT + k * CH, CH)],
                         cvs[b], sgs[b])

    def wait_g(k, b):
        sl = pl.ds(k * CH, CH)
        pltpu.make_async_copy(ls_hbm.at[isa.at[sl]], avs[b], sgs[b]).wait()
        pltpu.make_async_copy(ld_hbm.at[ida.at[sl]], bvs[b], sgs[b]).wait()
        pltpu.make_async_copy(wrl_hbm.at[pl.ds(wid * EPT + k * CH, CH)],
                              cvs[b], sgs[b]).wait()

    def out(k, b):
        off = wid * EPT + k * CH
        pltpu.async_copy(avs[b], enc_hbm.at[pl.ds(off, CH)], sos[b])
        pltpu.async_copy(uvs[b], t0_hbm.at[pl.ds(off, CH)], sos[b])
        pltpu.async_copy(vvs[b], t2_hbm.at[pl.ds(off, CH)], sos[b])

    def wait_out(k, b):
        off = wid * EPT + k * CH
        pltpu.make_async_copy(avs[b], enc_hbm.at[pl.ds(off, CH)],
                              sos[b]).wait()
        pltpu.make_async_copy(uvs[b], t0_hbm.at[pl.ds(off, CH)],
                              sos[b]).wait()
        pltpu.make_async_copy(vvs[b], t2_hbm.at[pl.ds(off, CH)],
                              sos[b]).wait()

    def compute(k, b):
        av, bv, cv, uv, vv = avs[b], bvs[b], cvs[b], uvs[b], vvs[b]

        def rstep(r, c2):
            for j in range(8):
                sl = pl.ds(16 * j, 16)
                av[r, sl] = av[r, sl] + bv[r, sl] + cv[r, sl]
            return c2

        lax.fori_loop(0, CH, rstep, 0)

        for i in range(CH // 16):
            sv = isa[pl.ds(k * CH + i * 16, 16)]
            dv = ida[pl.ds(k * CH + i * 16, 16)]
            uv[pl.ds(i * 16, 16)] = plsc.load_gather(cid_v, [sv])
            vv[pl.ds(i * 16, 16)] = plsc.load_gather(cid_v, [dv])

    issue(0, 0)

    def gstep(g, carry):
        k0 = 2 * g
        k1 = k0 + 1

        @pl.when(k0 >= 1)
        def _():
            wait_out(k0 - 1, 1)

        issue(k1, 1)
        wait_g(k0, 0)
        compute(k0, 0)
        out(k0, 0)

        @pl.when(k1 + 1 < NCH)
        def _():
            wait_out(k0, 0)
            issue(k1 + 1, 0)

        wait_g(k1, 1)
        compute(k1, 1)
        out(k1, 1)
        return carry

    lax.fori_loop(0, NCH // 2, gstep, 0)
    wait_out(NCH - 2, 0)
    wait_out(NCH - 1, 1)


_edge_out = pl.kernel(
    _edge_out_body,
    out_type=[
        jax.ShapeDtypeStruct((NE3, D), jnp.float32),
        jax.ShapeDtypeStruct((NE3,), jnp.int32),
        jax.ShapeDtypeStruct((NE3,), jnp.int32),
    ],
    mesh=_mesh,
    compiler_params=_sc_params,
    scratch_types=[
        pltpu.VMEM((EPT,), jnp.int32),
        pltpu.VMEM((EPT,), jnp.int32),
        pltpu.VMEM((NP,), jnp.int32),
        pltpu.VMEM((CH, D), jnp.float32),
        pltpu.VMEM((CH, D), jnp.float32),
        pltpu.VMEM((CH, D), jnp.float32),
        pltpu.VMEM((CH, D), jnp.float32),
        pltpu.VMEM((CH, D), jnp.float32),
        pltpu.VMEM((CH, D), jnp.float32),
        pltpu.VMEM((CH,), jnp.int32),
        pltpu.VMEM((CH,), jnp.int32),
        pltpu.VMEM((CH,), jnp.int32),
        pltpu.VMEM((CH,), jnp.int32),
        pltpu.SemaphoreType.DMA,
        pltpu.SemaphoreType.DMA,
        pltpu.SemaphoreType.DMA,
        pltpu.SemaphoreType.DMA,
    ],
)


def kernel(concept_ids, edge_index, edge_attr, concept_embedding,
           relation_embedding, W_msg, b_msg, W_self, W_upd, b_upd,
           W_lin, b_lin):
    src = edge_index[0]
    dst = edge_index[1]
    pad = NE3 - NE
    src1 = jnp.pad(src, (0, pad))
    dst1 = jnp.pad(dst, (0, pad), constant_values=NN)
    attr3 = jnp.pad(edge_attr, ((0, pad), (0, 0)))
    cid_pad = jnp.concatenate(
        [concept_ids, jnp.zeros((NP - NN,), jnp.int32)])
    rp = jnp.pad(relation_embedding, ((0, NRELP - NREL), (0, 0)))

    x = _gather_x(concept_embedding, cid_pad)
    xm, xs, relcat = _node_pre(x, W_msg, b_msg, W_self, rp, W_lin)
    wrm, wrl = _edge_bias(attr3, relcat)
    agg2 = _msg_agg(xm, wrm, src1, dst1)
    ls, ld = _node_upd(agg2, xs, W_upd, b_upd, W_lin, b_lin)
    enc, t0, t2 = _edge_out(ls, ld, wrl, src1, dst1, cid_pad)
    tid = jnp.stack(
        [t0[:NE], edge_attr[:, 0].astype(jnp.int32), t2[:NE]], axis=1)
    return enc[:NE], tid


# speedup vs baseline: 2.1413x; 2.1413x over previous
import jax
import jax.numpy as jnp
from jax import lax
from jax.experimental import pallas as pl
from jax.experimental.pallas import tpu as pltpu
from jax.experimental.pallas import tpu_sc as plsc

D = 128
NN = 10000
NE = 320000
NREL = 38
NRELP = 40
NP = 10240
NC, NS = 2, 16
NW = NC * NS
CH = 80
EPT = NE // NW
NCH = EPT // CH
APT = NP // NW

_mesh = plsc.VectorSubcoreMesh(
    core_axis_name="c", subcore_axis_name="s", num_cores=NC, num_subcores=NS)
_sc_params = pltpu.CompilerParams(needs_layout_passes=False)


def _wid():
    return lax.axis_index("s") * NC + lax.axis_index("c")


def _gather_x_body(ce_hbm, cid_hbm, x_hbm, idx_v, rows_v, sem):
    base = _wid() * APT

    def step(k, carry):
        off = base + k * CH
        pltpu.sync_copy(cid_hbm.at[pl.ds(off, CH)], idx_v)
        pltpu.async_copy(ce_hbm.at[idx_v], rows_v, sem).wait()
        pltpu.sync_copy(rows_v, x_hbm.at[pl.ds(off, CH)])
        return carry

    lax.fori_loop(0, APT // CH, step, 0)


_gather_x = pl.kernel(
    _gather_x_body,
    out_type=jax.ShapeDtypeStruct((NP, D), jnp.float32),
    mesh=_mesh,
    compiler_params=_sc_params,
    scratch_types=[
        pltpu.VMEM((CH,), jnp.int32),
        pltpu.VMEM((CH, D), jnp.float32),
        pltpu.SemaphoreType.DMA,
    ],
)


def _node_pre_body(x_ref, wm_ref, bm_ref, ws_ref, rp_ref, wl_ref,
                   xm_ref, xs_ref, relcat_ref):
    x = x_ref[...]
    xm_ref[...] = jnp.dot(x, wm_ref[0:D, :],
                          preferred_element_type=jnp.float32) + bm_ref[...]
    xs_ref[...] = jnp.dot(x, ws_ref[...], preferred_element_type=jnp.float32)
    rp = rp_ref[...]
    rm = jnp.dot(rp, wm_ref[D:2 * D, :], preferred_element_type=jnp.float32)
    rl = jnp.dot(rp, wl_ref[D:2 * D, :], preferred_element_type=jnp.float32)
    relcat_ref[...] = jnp.concatenate([rm, rl], axis=1)


_node_pre = pl.pallas_call(
    _node_pre_body,
    out_shape=[
        jax.ShapeDtypeStruct((NP, D), jnp.float32),
        jax.ShapeDtypeStruct((NP, D), jnp.float32),
        jax.ShapeDtypeStruct((NRELP, 2 * D), jnp.float32),
    ],
)


EB = 2000


def _edge_bias_body(attr_ref, relcat_ref, wrm_ref, wrl_ref):
    attr = attr_ref[...]
    reli = attr[:, 0:1].astype(jnp.int32)
    w = attr[:, 1:2]
    io = lax.broadcasted_iota(jnp.int32, (EB, NRELP), 1)
    ohw = jnp.where(reli == io, w, 0.0)
    big = jnp.dot(ohw, relcat_ref[...], preferred_element_type=jnp.float32)
    wrm_ref[...] = big[:, 0:D]
    wrl_ref[...] = big[:, D:2 * D]


_edge_bias = pl.pallas_call(
    _edge_bias_body,
    grid=(NE // EB,),
    in_specs=[
        pl.BlockSpec((EB, 2), lambda i: (i, 0)),
        pl.BlockSpec((NRELP, 2 * D), lambda i: (0, 0)),
    ],
    out_specs=[
        pl.BlockSpec((EB, D), lambda i: (i, 0)),
        pl.BlockSpec((EB, D), lambda i: (i, 0)),
    ],
    out_shape=[
        jax.ShapeDtypeStruct((NE, D), jnp.float32),
        jax.ShapeDtypeStruct((NE, D), jnp.float32),
    ],
)


def _msg_agg_body(xm_hbm, wrm_hbm, src_hbm, dst_hbm, agg_hbm,
                  is0, is1, id0, id1, r0, r1, w0, w1, shared,
                  sg0, sg1, sa0, sa1, si0, si1, sd0, sd1):
    c = lax.axis_index("c")
    s = lax.axis_index("s")
    wid = s * NC + c
    nstripe = NP // NS
    iss = (is0, is1)
    ids = (id0, id1)
    rows = (r0, r1)
    wrms = (w0, w1)
    sgs = (sg0, sg1)
    sas = (sa0, sa1)
    sis = (si0, si1)
    sds = (sd0, sd1)

    zero = jnp.zeros((16,), jnp.float32)
    for r in range(40):
        for j in range(8):
            r0[r, pl.ds(16 * j, 16)] = zero

    def zstep(i, carry):
        pltpu.sync_copy(r0.at[pl.ds(0, 40)],
                        shared.at[pl.ds(s * nstripe + i * 40, 40)])
        return carry

    lax.fori_loop(0, nstripe // 40, zstep, 0)
    plsc.subcore_barrier()

    def issue_is(k, b):
        pltpu.async_copy(src_hbm.at[pl.ds(wid * EPT + k * CH, CH)],
                         iss[b], sis[b])

    def wait_is(k, b):
        pltpu.make_async_copy(src_hbm.at[pl.ds(wid * EPT + k * CH, CH)],
                              iss[b], sis[b]).wait()

    def issue_id(k, b):
        pltpu.async_copy(dst_hbm.at[pl.ds(wid * EPT + k * CH, CH)],
                         ids[b], sds[b])

    def wait_id(k, b):
        pltpu.make_async_copy(dst_hbm.at[pl.ds(wid * EPT + k * CH, CH)],
                              ids[b], sds[b]).wait()

    def issue_g(k, b):
        pltpu.async_copy(xm_hbm.at[iss[b]], rows[b], sgs[b])
        pltpu.async_copy(wrm_hbm.at[pl.ds(wid * EPT + k * CH, CH)],
                         wrms[b], sgs[b])

    def wait_g(k, b):
        pltpu.make_async_copy(xm_hbm.at[iss[b]], rows[b], sgs[b]).wait()
        pltpu.make_async_copy(wrm_hbm.at[pl.ds(wid * EPT + k * CH, CH)],
                              wrms[b], sgs[b]).wait()

    def sadd(k, b):
        pltpu.async_copy(rows[b], shared.at[ids[b]], sas[b], add=True)

    def wait_sa(k, b):
        pltpu.make_async_copy(rows[b], shared.at[ids[b]], sas[b]).wait()

    def compute(b):
        rv, wv = rows[b], wrms[b]

        def rstep(r, c2):
            for j in range(8):
                sl = pl.ds(16 * j, 16)
                rv[r, sl] = jnp.maximum(rv[r, sl] + wv[r, sl], 0.0)
            return c2

        lax.fori_loop(0, CH, rstep, 0)

    pltpu.sync_copy(src_hbm.at[pl.ds(wid * EPT, CH)], is0)
    pltpu.sync_copy(dst_hbm.at[pl.ds(wid * EPT, CH)], id0)
    pltpu.sync_copy(src_hbm.at[pl.ds(wid * EPT + CH, CH)], is1)
    issue_g(0, 0)

    def gstep(g, carry):
        k0 = 2 * g
        k1 = k0 + 1

        wait_g(k0, 0)
        compute(0)

        @pl.when(k0 >= 1)
        def _():
            wait_id(k0, 0)

        sadd(k0, 0)

        @pl.when(k0 >= 1)
        def _():
            wait_sa(k0 - 1, 1)
            wait_is(k1, 1)

        issue_g(k1, 1)
        issue_id(k1, 1)

        @pl.when(k0 + 2 < NCH)
        def _():
            issue_is(k0 + 2, 0)

        wait_g(k1, 1)
        compute(1)
        wait_id(k1, 1)
        sadd(k1, 1)

        @pl.when(k1 + 1 < NCH)
        def _():
            wait_sa(k0, 0)
            wait_is(k1 + 1, 0)
            issue_g(k1 + 1, 0)
            issue_id(k1 + 1, 0)

            @pl.when(k1 + 2 < NCH)
            def _():
                issue_is(k1 + 2, 1)

        return carry

    lax.fori_loop(0, (NCH - 1) // 2, gstep, 0)
    wait_g(NCH - 1, 0)
    compute(0)
    wait_id(NCH - 1, 0)
    sadd(NCH - 1, 0)
    wait_sa(NCH - 2, 1)
    wait_sa(NCH - 1, 0)
    plsc.subcore_barrier()
    pltpu.sync_copy(shared.at[pl.ds(s * nstripe, nstripe)],
                    agg_hbm.at[c, pl.ds(s * nstripe, nstripe)])


_msg_agg = pl.kernel(
    _msg_agg_body,
    out_type=jax.ShapeDtypeStruct((NC, NP, D), jnp.float32),
    mesh=_mesh,
    compiler_params=_sc_params,
    scratch_types=[
        pltpu.VMEM((CH,), jnp.int32),
        pltpu.VMEM((CH,), jnp.int32),
        pltpu.VMEM((CH,), jnp.int32),
        pltpu.VMEM((CH,), jnp.int32),
        pltpu.VMEM((CH, D), jnp.float32),
        pltpu.VMEM((CH, D), jnp.float32),
        pltpu.VMEM((CH, D), jnp.float32),
        pltpu.VMEM((CH, D), jnp.float32),
        pltpu.VMEM_SHARED((NP, D), jnp.float32),
        pltpu.SemaphoreType.DMA,
        pltpu.SemaphoreType.DMA,
        pltpu.SemaphoreType.DMA,
        pltpu.SemaphoreType.DMA,
        pltpu.SemaphoreType.DMA,
        pltpu.SemaphoreType.DMA,
        pltpu.SemaphoreType.DMA,
        pltpu.SemaphoreType.DMA,
    ],
)


def _node_upd_body(agg_ref, xs_ref, wu_ref, bu_ref, wl_ref, bl_ref,
                   ls_ref, ld_ref):
    aggs = agg_ref[0] + agg_ref[1]
    x2 = jnp.maximum(
        jnp.dot(aggs, wu_ref[...], preferred_element_type=jnp.float32)
        + xs_ref[...] + bu_ref[...], 0.0)
    ls_ref[...] = jnp.dot(x2, wl_ref[0:D, :],
                          preferred_element_type=jnp.float32) + bl_ref[...]
    ld_ref[...] = jnp.dot(x2, wl_ref[2 * D:3 * D, :],
                          preferred_element_type=jnp.float32)


_node_upd = pl.pallas_call(
    _node_upd_body,
    out_shape=[
        jax.ShapeDtypeStruct((NP, D), jnp.float32),
        jax.ShapeDtypeStruct((NP, D), jnp.float32),
    ],
)


def _edge_out_body(ls_hbm, ld_hbm, wrl_hbm, src_hbm, dst_hbm,
                   cid_hbm, enc_hbm, t0_hbm, t2_hbm,
                   isa, ida, cid_v,
                   a0, a1, b0, b1, c0, c1, u0, u1, v0, v1,
                   sg0, sg1, so0, so1):
    wid = _wid()
    avs = (a0, a1)
    bvs = (b0, b1)
    cvs = (c0, c1)
    uvs = (u0, u1)
    vvs = (v0, v1)
    sgs = (sg0, sg1)
    sos = (so0, so1)

    pltpu.sync_copy(src_hbm.at[pl.ds(wid * EPT, EPT)], isa)
    pltpu.sync_copy(dst_hbm.at[pl.ds(wid * EPT, EPT)], ida)
    pltpu.sync_copy(cid_hbm, cid_v)

    def issue(k, b):
        sl = pl.ds(k * CH, CH)
        pltpu.async_copy(ls_hbm.at[isa.at[sl]], avs[b], sgs[b])
        pltpu.async_copy(ld_hbm.at[ida.at[sl]], bvs[b], sgs[b])
        pltpu.async_copy(wrl_hbm.at[pl.ds(wid * EPT + k * CH, CH)],
                         cvs[b], sgs[b])

    def wait_g(k, b):
        sl = pl.ds(k * CH, CH)
        pltpu.make_async_copy(ls_hbm.at[isa.at[sl]], avs[b], sgs[b]).wait()
        pltpu.make_async_copy(ld_hbm.at[ida.at[sl]], bvs[b], sgs[b]).wait()
        pltpu.make_async_copy(wrl_hbm.at[pl.ds(wid * EPT + k * CH, CH)],
                              cvs[b], sgs[b]).wait()

    def out(k, b):
        off = wid * EPT + k * CH
        pltpu.async_copy(avs[b], enc_hbm.at[pl.ds(off, CH)], sos[b])
        pltpu.async_copy(uvs[b], t0_hbm.at[pl.ds(off, CH)], sos[b])
        pltpu.async_copy(vvs[b], t2_hbm.at[pl.ds(off, CH)], sos[b])

    def wait_out(k, b):
        off = wid * EPT + k * CH
        pltpu.make_async_copy(avs[b], enc_hbm.at[pl.ds(off, CH)],
                              sos[b]).wait()
        pltpu.make_async_copy(uvs[b], t0_hbm.at[pl.ds(off, CH)],
                              sos[b]).wait()
        pltpu.make_async_copy(vvs[b], t2_hbm.at[pl.ds(off, CH)],
                              sos[b]).wait()

    def compute(k, b):
        av, bv, cv, uv, vv = avs[b], bvs[b], cvs[b], uvs[b], vvs[b]

        def rstep(r, c2):
            for j in range(8):
                sl = pl.ds(16 * j, 16)
                av[r, sl] = av[r, sl] + bv[r, sl] + cv[r, sl]
            return c2

        lax.fori_loop(0, CH, rstep, 0)

        for i in range(CH // 16):
            sv = isa[pl.ds(k * CH + i * 16, 16)]
            dv = ida[pl.ds(k * CH + i * 16, 16)]
            uv[pl.ds(i * 16, 16)] = plsc.load_gather(cid_v, [sv])
            vv[pl.ds(i * 16, 16)] = plsc.load_gather(cid_v, [dv])

    issue(0, 0)

    def gstep(g, carry):
        k0 = 2 * g
        k1 = k0 + 1

        @pl.when(k0 >= 1)
        def _():
            wait_out(k0 - 1, 1)

        issue(k1, 1)
        wait_g(k0, 0)
        compute(k0, 0)
        out(k0, 0)

        @pl.when(k1 + 1 < NCH)
        def _():
            wait_out(k0, 0)
            issue(k1 + 1, 0)

        wait_g(k1, 1)
        compute(k1, 1)
        out(k1, 1)
        return carry

    lax.fori_loop(0, (NCH - 1) // 2, gstep, 0)
    wait_g(NCH - 1, 0)
    compute(NCH - 1, 0)
    out(NCH - 1, 0)
    wait_out(NCH - 2, 1)
    wait_out(NCH - 1, 0)


_edge_out = pl.kernel(
    _edge_out_body,
    out_type=[
        jax.ShapeDtypeStruct((NE, D), jnp.float32),
        jax.ShapeDtypeStruct((NE,), jnp.int32),
        jax.ShapeDtypeStruct((NE,), jnp.int32),
    ],
    mesh=_mesh,
    compiler_params=_sc_params,
    scratch_types=[
        pltpu.VMEM((EPT,), jnp.int32),
        pltpu.VMEM((EPT,), jnp.int32),
        pltpu.VMEM((NP,), jnp.int32),
        pltpu.VMEM((CH, D), jnp.float32),
        pltpu.VMEM((CH, D), jnp.float32),
        pltpu.VMEM((CH, D), jnp.float32),
        pltpu.VMEM((CH, D), jnp.float32),
        pltpu.VMEM((CH, D), jnp.float32),
        pltpu.VMEM((CH, D), jnp.float32),
        pltpu.VMEM((CH,), jnp.int32),
        pltpu.VMEM((CH,), jnp.int32),
        pltpu.VMEM((CH,), jnp.int32),
        pltpu.VMEM((CH,), jnp.int32),
        pltpu.SemaphoreType.DMA,
        pltpu.SemaphoreType.DMA,
        pltpu.SemaphoreType.DMA,
        pltpu.SemaphoreType.DMA,
    ],
)


def kernel(concept_ids, edge_index, edge_attr, concept_embedding,
           relation_embedding, W_msg, b_msg, W_self, W_upd, b_upd,
           W_lin, b_lin):
    src1 = edge_index[0]
    dst1 = edge_index[1]
    cid_pad = jnp.concatenate(
        [concept_ids, jnp.zeros((NP - NN,), jnp.int32)])
    rp = jnp.pad(relation_embedding, ((0, NRELP - NREL), (0, 0)))

    x = _gather_x(concept_embedding, cid_pad)
    xm, xs, relcat = _node_pre(x, W_msg, b_msg, W_self, rp, W_lin)
    wrm, wrl = _edge_bias(edge_attr, relcat)
    agg2 = _msg_agg(xm, wrm, src1, dst1)
    ls, ld = _node_upd(agg2, xs, W_upd, b_upd, W_lin, b_lin)
    enc, t0, t2 = _edge_out(ls, ld, wrl, src1, dst1, cid_pad)
    tid = jnp.stack(
        [t0, edge_attr[:, 0].astype(jnp.int32), t2], axis=1)
    return enc, tid


# speedup vs baseline: 2.1600x; 1.0087x over previous
import jax
import jax.numpy as jnp
from jax import lax
from jax.experimental import pallas as pl
from jax.experimental.pallas import tpu as pltpu
from jax.experimental.pallas import tpu_sc as plsc

D = 128
NN = 10000
NE = 320000
NREL = 38
NRELP = 40
NP = 10240
NC, NS = 2, 16
NW = NC * NS
CH = 80
EPT = NE // NW
NCH = EPT // CH
APT = NP // NW

_mesh = plsc.VectorSubcoreMesh(
    core_axis_name="c", subcore_axis_name="s", num_cores=NC, num_subcores=NS)
_sc_params = pltpu.CompilerParams(needs_layout_passes=False)


def _wid():
    return lax.axis_index("s") * NC + lax.axis_index("c")


def _gather_x_body(ce_hbm, cid_hbm, x_hbm, idx_v, rows_v, sem):
    base = _wid() * APT

    def step(k, carry):
        off = base + k * CH
        pltpu.sync_copy(cid_hbm.at[pl.ds(off, CH)], idx_v)
        pltpu.async_copy(ce_hbm.at[idx_v], rows_v, sem).wait()
        pltpu.sync_copy(rows_v, x_hbm.at[pl.ds(off, CH)])
        return carry

    lax.fori_loop(0, APT // CH, step, 0)


_gather_x = pl.kernel(
    _gather_x_body,
    out_type=jax.ShapeDtypeStruct((NP, D), jnp.float32),
    mesh=_mesh,
    compiler_params=_sc_params,
    scratch_types=[
        pltpu.VMEM((CH,), jnp.int32),
        pltpu.VMEM((CH, D), jnp.float32),
        pltpu.SemaphoreType.DMA,
    ],
)


def _node_pre_body(x_ref, wm_ref, bm_ref, ws_ref, rp_ref, wl_ref,
                   xm_ref, xs_ref, relcat_ref):
    x = x_ref[...]
    xm_ref[...] = jnp.dot(x, wm_ref[0:D, :],
                          preferred_element_type=jnp.float32) + bm_ref[...]
    xs_ref[...] = jnp.dot(x, ws_ref[...], preferred_element_type=jnp.float32)
    rp = rp_ref[...]
    rm = jnp.dot(rp, wm_ref[D:2 * D, :], preferred_element_type=jnp.float32)
    rl = jnp.dot(rp, wl_ref[D:2 * D, :], preferred_element_type=jnp.float32)
    relcat_ref[...] = jnp.concatenate([rm, rl], axis=1)


_node_pre = pl.pallas_call(
    _node_pre_body,
    out_shape=[
        jax.ShapeDtypeStruct((NP, D), jnp.float32),
        jax.ShapeDtypeStruct((NP, D), jnp.float32),
        jax.ShapeDtypeStruct((NRELP, 2 * D), jnp.float32),
    ],
)


EB = 2000


def _pack_bf16(lo, hi):
    rl = lo.astype(jnp.bfloat16).astype(jnp.float32)
    rh = hi.astype(jnp.bfloat16).astype(jnp.float32)
    il = lax.shift_right_logical(
        lax.bitcast_convert_type(rl, jnp.int32), 16)
    ih = lax.bitcast_convert_type(rh, jnp.int32)
    return il | ih


def _edge_bias_body(attr_ref, relcat_ref, wrm_ref, wrl_ref):
    attr = attr_ref[...]
    reli = attr[:, 0:1].astype(jnp.int32)
    w = attr[:, 1:2]
    io = lax.broadcasted_iota(jnp.int32, (EB, NRELP), 1)
    ohw = jnp.where(reli == io, w, 0.0)
    big = jnp.dot(ohw, relcat_ref[...], preferred_element_type=jnp.float32)
    wrm_ref[...] = _pack_bf16(big[:, 0:D // 2], big[:, D // 2:D])
    wrl_ref[...] = _pack_bf16(big[:, D:D + D // 2], big[:, D + D // 2:2 * D])


_edge_bias = pl.pallas_call(
    _edge_bias_body,
    grid=(NE // EB,),
    in_specs=[
        pl.BlockSpec((EB, 2), lambda i: (i, 0)),
        pl.BlockSpec((NRELP, 2 * D), lambda i: (0, 0)),
    ],
    out_specs=[
        pl.BlockSpec((EB, D // 2), lambda i: (i, 0)),
        pl.BlockSpec((EB, D // 2), lambda i: (i, 0)),
    ],
    out_shape=[
        jax.ShapeDtypeStruct((NE, D // 2), jnp.int32),
        jax.ShapeDtypeStruct((NE, D // 2), jnp.int32),
    ],
)


def _msg_agg_body(xm_hbm, wrm_hbm, src_hbm, dst_hbm, agg_hbm,
                  is0, is1, id0, id1, r0, r1, w0, w1, shared,
                  sg0, sg1, sa0, sa1, si0, si1, sd0, sd1):
    c = lax.axis_index("c")
    s = lax.axis_index("s")
    wid = s * NC + c
    nstripe = NP // NS
    iss = (is0, is1)
    ids = (id0, id1)
    rows = (r0, r1)
    wrms = (w0, w1)
    sgs = (sg0, sg1)
    sas = (sa0, sa1)
    sis = (si0, si1)
    sds = (sd0, sd1)

    zero = jnp.zeros((16,), jnp.float32)
    for r in range(40):
        for j in range(8):
            r0[r, pl.ds(16 * j, 16)] = zero

    def zstep(i, carry):
        pltpu.sync_copy(r0.at[pl.ds(0, 40)],
                        shared.at[pl.ds(s * nstripe + i * 40, 40)])
        return carry

    lax.fori_loop(0, nstripe // 40, zstep, 0)
    plsc.subcore_barrier()

    def issue_is(k, b):
        pltpu.async_copy(src_hbm.at[pl.ds(wid * EPT + k * CH, CH)],
                         iss[b], sis[b])

    def wait_is(k, b):
        pltpu.make_async_copy(src_hbm.at[pl.ds(wid * EPT + k * CH, CH)],
                              iss[b], sis[b]).wait()

    def issue_id(k, b):
        pltpu.async_copy(dst_hbm.at[pl.ds(wid * EPT + k * CH, CH)],
                         ids[b], sds[b])

    def wait_id(k, b):
        pltpu.make_async_copy(dst_hbm.at[pl.ds(wid * EPT + k * CH, CH)],
                              ids[b], sds[b]).wait()

    MASKHI = jnp.full((16,), -65536, jnp.int32)

    def issue_g(k, b):
        pltpu.async_copy(xm_hbm.at[iss[b]], rows[b], sgs[b])
        pltpu.async_copy(wrm_hbm.at[pl.ds(wid * EPT + k * CH, CH)],
                         wrms[b], sgs[b])

    def wait_g(k, b):
        pltpu.make_async_copy(xm_hbm.at[iss[b]], rows[b], sgs[b]).wait()
        pltpu.make_async_copy(wrm_hbm.at[pl.ds(wid * EPT + k * CH, CH)],
                              wrms[b], sgs[b]).wait()

    def sadd(k, b):
        pltpu.async_copy(rows[b], shared.at[ids[b]], sas[b], add=True)

    def wait_sa(k, b):
        pltpu.make_async_copy(rows[b], shared.at[ids[b]], sas[b]).wait()

    def compute(b):
        rv, wv = rows[b], wrms[b]

        def rstep(r, c2):
            for g in range(4):
                pk = wv[r, pl.ds(16 * g, 16)]
                lo = plsc.bitcast(pk << 16, jnp.float32)
                hi = plsc.bitcast(pk & MASKHI, jnp.float32)
                sl = pl.ds(16 * g, 16)
                sh = pl.ds(16 * g + D // 2, 16)
                rv[r, sl] = jnp.maximum(rv[r, sl] + lo, 0.0)
                rv[r, sh] = jnp.maximum(rv[r, sh] + hi, 0.0)
            return c2

        lax.fori_loop(0, CH, rstep, 0)

    pltpu.sync_copy(src_hbm.at[pl.ds(wid * EPT, CH)], is0)
    pltpu.sync_copy(dst_hbm.at[pl.ds(wid * EPT, CH)], id0)
    pltpu.sync_copy(src_hbm.at[pl.ds(wid * EPT + CH, CH)], is1)
    issue_g(0, 0)

    def gstep(g, carry):
        k0 = 2 * g
        k1 = k0 + 1

        wait_g(k0, 0)
        compute(0)

        @pl.when(k0 >= 1)
        def _():
            wait_id(k0, 0)

        sadd(k0, 0)

        @pl.when(k0 >= 1)
        def _():
            wait_sa(k0 - 1, 1)
            wait_is(k1, 1)

        issue_g(k1, 1)
        issue_id(k1, 1)

        @pl.when(k0 + 2 < NCH)
        def _():
            issue_is(k0 + 2, 0)

        wait_g(k1, 1)
        compute(1)
        wait_id(k1, 1)
        sadd(k1, 1)

        @pl.when(k1 + 1 < NCH)
        def _():
            wait_sa(k0, 0)
            wait_is(k1 + 1, 0)
            issue_g(k1 + 1, 0)
            issue_id(k1 + 1, 0)

            @pl.when(k1 + 2 < NCH)
            def _():
                issue_is(k1 + 2, 1)

        return carry

    lax.fori_loop(0, (NCH - 1) // 2, gstep, 0)
    wait_g(NCH - 1, 0)
    compute(0)
    wait_id(NCH - 1, 0)
    sadd(NCH - 1, 0)
    wait_sa(NCH - 2, 1)
    wait_sa(NCH - 1, 0)
    plsc.subcore_barrier()
    pltpu.sync_copy(shared.at[pl.ds(s * nstripe, nstripe)],
                    agg_hbm.at[c, pl.ds(s * nstripe, nstripe)])


_msg_agg = pl.kernel(
    _msg_agg_body,
    out_type=jax.ShapeDtypeStruct((NC, NP, D), jnp.float32),
    mesh=_mesh,
    compiler_params=_sc_params,
    scratch_types=[
        pltpu.VMEM((CH,), jnp.int32),
        pltpu.VMEM((CH,), jnp.int32),
        pltpu.VMEM((CH,), jnp.int32),
        pltpu.VMEM((CH,), jnp.int32),
        pltpu.VMEM((CH, D), jnp.float32),
        pltpu.VMEM((CH, D), jnp.float32),
        pltpu.VMEM((CH, D // 2), jnp.int32),
        pltpu.VMEM((CH, D // 2), jnp.int32),
        pltpu.VMEM_SHARED((NP, D), jnp.float32),
        pltpu.SemaphoreType.DMA,
        pltpu.SemaphoreType.DMA,
        pltpu.SemaphoreType.DMA,
        pltpu.SemaphoreType.DMA,
        pltpu.SemaphoreType.DMA,
        pltpu.SemaphoreType.DMA,
        pltpu.SemaphoreType.DMA,
        pltpu.SemaphoreType.DMA,
    ],
)


def _node_upd_body(agg_ref, xs_ref, wu_ref, bu_ref, wl_ref, bl_ref,
                   ls_ref, ld_ref):
    aggs = agg_ref[0] + agg_ref[1]
    x2 = jnp.maximum(
        jnp.dot(aggs, wu_ref[...], preferred_element_type=jnp.float32)
        + xs_ref[...] + bu_ref[...], 0.0)
    ls_ref[...] = jnp.dot(x2, wl_ref[0:D, :],
                          preferred_element_type=jnp.float32) + bl_ref[...]
    ld_ref[...] = jnp.dot(x2, wl_ref[2 * D:3 * D, :],
                          preferred_element_type=jnp.float32)


_node_upd = pl.pallas_call(
    _node_upd_body,
    out_shape=[
        jax.ShapeDtypeStruct((NP, D), jnp.float32),
        jax.ShapeDtypeStruct((NP, D), jnp.float32),
    ],
)


def _edge_out_body(ls_hbm, ld_hbm, wrl_hbm, src_hbm, dst_hbm,
                   cid_hbm, enc_hbm, t0_hbm, t2_hbm,
                   isa, ida, cid_v,
                   a0, a1, b0, b1, c0, c1, u0, u1, v0, v1,
                   sg0, sg1, so0, so1):
    wid = _wid()
    avs = (a0, a1)
    bvs = (b0, b1)
    cvs = (c0, c1)
    uvs = (u0, u1)
    vvs = (v0, v1)
    sgs = (sg0, sg1)
    sos = (so0, so1)

    pltpu.sync_copy(src_hbm.at[pl.ds(wid * EPT, EPT)], isa)
    pltpu.sync_copy(dst_hbm.at[pl.ds(wid * EPT, EPT)], ida)
    pltpu.sync_copy(cid_hbm, cid_v)

    def issue(k, b):
        sl = pl.ds(k * CH, CH)
        pltpu.async_copy(ls_hbm.at[isa.at[sl]], avs[b], sgs[b])
        pltpu.async_copy(ld_hbm.at[ida.at[sl]], bvs[b], sgs[b])
        pltpu.async_copy(wrl_hbm.at[pl.ds(wid * EPT + k * CH, CH)],
                         cvs[b], sgs[b])

    def wait_g(k, b):
        sl = pl.ds(k * CH, CH)
        pltpu.make_async_copy(ls_hbm.at[isa.at[sl]], avs[b], sgs[b]).wait()
        pltpu.make_async_copy(ld_hbm.at[ida.at[sl]], bvs[b], sgs[b]).wait()
        pltpu.make_async_copy(wrl_hbm.at[pl.ds(wid * EPT + k * CH, CH)],
                              cvs[b], sgs[b]).wait()

    def out(k, b):
        off = wid * EPT + k * CH
        pltpu.async_copy(avs[b], enc_hbm.at[pl.ds(off, CH)], sos[b])
        pltpu.async_copy(uvs[b], t0_hbm.at[pl.ds(off, CH)], sos[b])
        pltpu.async_copy(vvs[b], t2_hbm.at[pl.ds(off, CH)], sos[b])

    def wait_out(k, b):
        off = wid * EPT + k * CH
        pltpu.make_async_copy(avs[b], enc_hbm.at[pl.ds(off, CH)],
                              sos[b]).wait()
        pltpu.make_async_copy(uvs[b], t0_hbm.at[pl.ds(off, CH)],
                              sos[b]).wait()
        pltpu.make_async_copy(vvs[b], t2_hbm.at[pl.ds(off, CH)],
                              sos[b]).wait()

    MASKHI = jnp.full((16,), -65536, jnp.int32)

    def compute(k, b):
        av, bv, cv, uv, vv = avs[b], bvs[b], cvs[b], uvs[b], vvs[b]

        def rstep(r, c2):
            for g in range(4):
                pk = cv[r, pl.ds(16 * g, 16)]
                lo = plsc.bitcast(pk << 16, jnp.float32)
                hi = plsc.bitcast(pk & MASKHI, jnp.float32)
                sl = pl.ds(16 * g, 16)
                sh = pl.ds(16 * g + D // 2, 16)
                av[r, sl] = av[r, sl] + bv[r, sl] + lo
                av[r, sh] = av[r, sh] + bv[r, sh] + hi
            return c2

        lax.fori_loop(0, CH, rstep, 0)

        for i in range(CH // 16):
            sv = isa[pl.ds(k * CH + i * 16, 16)]
            dv = ida[pl.ds(k * CH + i * 16, 16)]
            uv[pl.ds(i * 16, 16)] = plsc.load_gather(cid_v, [sv])
            vv[pl.ds(i * 16, 16)] = plsc.load_gather(cid_v, [dv])

    issue(0, 0)

    def gstep(g, carry):
        k0 = 2 * g
        k1 = k0 + 1

        @pl.when(k0 >= 1)
        def _():
            wait_out(k0 - 1, 1)

        issue(k1, 1)
        wait_g(k0, 0)
        compute(k0, 0)
        out(k0, 0)

        @pl.when(k1 + 1 < NCH)
        def _():
            wait_out(k0, 0)
            issue(k1 + 1, 0)

        wait_g(k1, 1)
        compute(k1, 1)
        out(k1, 1)
        return carry

    lax.fori_loop(0, (NCH - 1) // 2, gstep, 0)
    wait_g(NCH - 1, 0)
    compute(NCH - 1, 0)
    out(NCH - 1, 0)
    wait_out(NCH - 2, 1)
    wait_out(NCH - 1, 0)


_edge_out = pl.kernel(
    _edge_out_body,
    out_type=[
        jax.ShapeDtypeStruct((NE, D), jnp.float32),
        jax.ShapeDtypeStruct((NE,), jnp.int32),
        jax.ShapeDtypeStruct((NE,), jnp.int32),
    ],
    mesh=_mesh,
    compiler_params=_sc_params,
    scratch_types=[
        pltpu.VMEM((EPT,), jnp.int32),
        pltpu.VMEM((EPT,), jnp.int32),
        pltpu.VMEM((NP,), jnp.int32),
        pltpu.VMEM((CH, D), jnp.float32),
        pltpu.VMEM((CH, D), jnp.float32),
        pltpu.VMEM((CH, D), jnp.float32),
        pltpu.VMEM((CH, D), jnp.float32),
        pltpu.VMEM((CH, D // 2), jnp.int32),
        pltpu.VMEM((CH, D // 2), jnp.int32),
        pltpu.VMEM((CH,), jnp.int32),
        pltpu.VMEM((CH,), jnp.int32),
        pltpu.VMEM((CH,), jnp.int32),
        pltpu.VMEM((CH,), jnp.int32),
        pltpu.SemaphoreType.DMA,
        pltpu.SemaphoreType.DMA,
        pltpu.SemaphoreType.DMA,
        pltpu.SemaphoreType.DMA,
    ],
)


def kernel(concept_ids, edge_index, edge_attr, concept_embedding,
           relation_embedding, W_msg, b_msg, W_self, W_upd, b_upd,
           W_lin, b_lin):
    src1 = edge_index[0]
    dst1 = edge_index[1]
    cid_pad = jnp.concatenate(
        [concept_ids, jnp.zeros((NP - NN,), jnp.int32)])
    rp = jnp.pad(relation_embedding, ((0, NRELP - NREL), (0, 0)))

    x = _gather_x(concept_embedding, cid_pad)
    xm, xs, relcat = _node_pre(x, W_msg, b_msg, W_self, rp, W_lin)
    wrm, wrl = _edge_bias(edge_attr, relcat)
    agg2 = _msg_agg(xm, wrm, src1, dst1)
    ls, ld = _node_upd(agg2, xs, W_upd, b_upd, W_lin, b_lin)
    enc, t0, t2 = _edge_out(ls, ld, wrl, src1, dst1, cid_pad)
    tid = jnp.stack(
        [t0, edge_attr[:, 0].astype(jnp.int32), t2], axis=1)
    return enc, tid


# speedup vs baseline: 2.2026x; 1.0197x over previous
import jax
import jax.numpy as jnp
from jax import lax
from jax.experimental import pallas as pl
from jax.experimental.pallas import tpu as pltpu
from jax.experimental.pallas import tpu_sc as plsc

D = 128
NN = 10000
NE = 320000
NREL = 38
NRELP = 40
NP = 10240
NC, NS = 2, 16
NW = NC * NS
CH = 80
EPT = NE // NW
NCH = EPT // CH
APT = NP // NW

_mesh = plsc.VectorSubcoreMesh(
    core_axis_name="c", subcore_axis_name="s", num_cores=NC, num_subcores=NS)
_sc_params = pltpu.CompilerParams(needs_layout_passes=False)


def _wid():
    return lax.axis_index("s") * NC + lax.axis_index("c")


def _gather_x_body(ce_hbm, cid_hbm, x_hbm, idx_v, rows_v, sem):
    base = _wid() * APT

    def step(k, carry):
        off = base + k * CH
        pltpu.sync_copy(cid_hbm.at[pl.ds(off, CH)], idx_v)
        pltpu.async_copy(ce_hbm.at[idx_v], rows_v, sem).wait()
        pltpu.sync_copy(rows_v, x_hbm.at[pl.ds(off, CH)])
        return carry

    lax.fori_loop(0, APT // CH, step, 0)


_gather_x = pl.kernel(
    _gather_x_body,
    out_type=jax.ShapeDtypeStruct((NP, D), jnp.float32),
    mesh=_mesh,
    compiler_params=_sc_params,
    scratch_types=[
        pltpu.VMEM((CH,), jnp.int32),
        pltpu.VMEM((CH, D), jnp.float32),
        pltpu.SemaphoreType.DMA,
    ],
)


def _node_pre_body(x_ref, wm_ref, bm_ref, ws_ref, rp_ref, wl_ref,
                   xm_ref, xs_ref, relcat_ref):
    x = x_ref[...]
    xm_ref[...] = jnp.dot(x, wm_ref[0:D, :],
                          preferred_element_type=jnp.float32) + bm_ref[...]
    xs_ref[...] = jnp.dot(x, ws_ref[...], preferred_element_type=jnp.float32)
    rp = rp_ref[...]
    rm = jnp.dot(rp, wm_ref[D:2 * D, :], preferred_element_type=jnp.float32)
    rl = jnp.dot(rp, wl_ref[D:2 * D, :], preferred_element_type=jnp.float32)
    relcat_ref[...] = jnp.concatenate([rm, rl], axis=1)


_node_pre = pl.pallas_call(
    _node_pre_body,
    out_shape=[
        jax.ShapeDtypeStruct((NP, D), jnp.float32),
        jax.ShapeDtypeStruct((NP, D), jnp.float32),
        jax.ShapeDtypeStruct((NRELP, 2 * D), jnp.float32),
    ],
)


EB = 2000


def _pack_bf16(lo, hi):
    rl = lo.astype(jnp.bfloat16).astype(jnp.float32)
    rh = hi.astype(jnp.bfloat16).astype(jnp.float32)
    il = lax.shift_right_logical(
        lax.bitcast_convert_type(rl, jnp.int32), 16)
    ih = lax.bitcast_convert_type(rh, jnp.int32)
    return il | ih


def _edge_bias_body(attr_ref, tab_ref, out_ref):
    attr = attr_ref[...]
    reli = attr[:, 0:1].astype(jnp.int32)
    w = attr[:, 1:2]
    io = lax.broadcasted_iota(jnp.int32, (EB, NRELP), 1)
    ohw = jnp.where(reli == io, w, 0.0)
    big = jnp.dot(ohw, tab_ref[...], preferred_element_type=jnp.float32)
    out_ref[...] = _pack_bf16(big[:, 0:D // 2], big[:, D // 2:D])


_edge_bias = pl.pallas_call(
    _edge_bias_body,
    grid=(NE // EB,),
    in_specs=[
        pl.BlockSpec((EB, 2), lambda i: (i, 0)),
        pl.BlockSpec((NRELP, D), lambda i: (0, 0)),
    ],
    out_specs=pl.BlockSpec((EB, D // 2), lambda i: (i, 0)),
    out_shape=jax.ShapeDtypeStruct((NE, D // 2), jnp.int32),
)


def _msg_agg_body(xm_hbm, wrm_hbm, src_hbm, dst_hbm, agg_hbm,
                  is0, is1, id0, id1, r0, r1, w0, w1, shared,
                  sg0, sg1, sa0, sa1, si0, si1, sd0, sd1):
    c = lax.axis_index("c")
    s = lax.axis_index("s")
    wid = s * NC + c
    nstripe = NP // NS
    iss = (is0, is1)
    ids = (id0, id1)
    rows = (r0, r1)
    wrms = (w0, w1)
    sgs = (sg0, sg1)
    sas = (sa0, sa1)
    sis = (si0, si1)
    sds = (sd0, sd1)

    zero = jnp.zeros((16,), jnp.float32)
    for r in range(40):
        for j in range(8):
            r0[r, pl.ds(16 * j, 16)] = zero

    def zstep(i, carry):
        pltpu.sync_copy(r0.at[pl.ds(0, 40)],
                        shared.at[pl.ds(s * nstripe + i * 40, 40)])
        return carry

    lax.fori_loop(0, nstripe // 40, zstep, 0)
    plsc.subcore_barrier()

    def issue_is(k, b):
        pltpu.async_copy(src_hbm.at[pl.ds(wid * EPT + k * CH, CH)],
                         iss[b], sis[b])

    def wait_is(k, b):
        pltpu.make_async_copy(src_hbm.at[pl.ds(wid * EPT + k * CH, CH)],
                              iss[b], sis[b]).wait()

    def issue_id(k, b):
        pltpu.async_copy(dst_hbm.at[pl.ds(wid * EPT + k * CH, CH)],
                         ids[b], sds[b])

    def wait_id(k, b):
        pltpu.make_async_copy(dst_hbm.at[pl.ds(wid * EPT + k * CH, CH)],
                              ids[b], sds[b]).wait()

    MASKHI = jnp.full((16,), -65536, jnp.int32)

    def issue_g(k, b):
        pltpu.async_copy(xm_hbm.at[iss[b]], rows[b], sgs[b])
        pltpu.async_copy(wrm_hbm.at[pl.ds(wid * EPT + k * CH, CH)],
                         wrms[b], sgs[b])

    def wait_g(k, b):
        pltpu.make_async_copy(xm_hbm.at[iss[b]], rows[b], sgs[b]).wait()
        pltpu.make_async_copy(wrm_hbm.at[pl.ds(wid * EPT + k * CH, CH)],
                              wrms[b], sgs[b]).wait()

    def sadd(k, b):
        pltpu.async_copy(rows[b], shared.at[ids[b]], sas[b], add=True)

    def wait_sa(k, b):
        pltpu.make_async_copy(rows[b], shared.at[ids[b]], sas[b]).wait()

    def compute(b):
        rv, wv = rows[b], wrms[b]

        def rstep(ri, c2):
            for u in range(4):
                r = 4 * ri + u
                for g in range(4):
                    pk = wv[r, pl.ds(16 * g, 16)]
                    lo = plsc.bitcast(pk << 16, jnp.float32)
                    hi = plsc.bitcast(pk & MASKHI, jnp.float32)
                    sl = pl.ds(16 * g, 16)
                    sh = pl.ds(16 * g + D // 2, 16)
                    rv[r, sl] = jnp.maximum(rv[r, sl] + lo, 0.0)
                    rv[r, sh] = jnp.maximum(rv[r, sh] + hi, 0.0)
            return c2

        lax.fori_loop(0, CH // 4, rstep, 0)

    pltpu.sync_copy(src_hbm.at[pl.ds(wid * EPT, CH)], is0)
    pltpu.sync_copy(dst_hbm.at[pl.ds(wid * EPT, CH)], id0)
    pltpu.sync_copy(src_hbm.at[pl.ds(wid * EPT + CH, CH)], is1)
    issue_g(0, 0)

    def gstep(g, carry):
        k0 = 2 * g
        k1 = k0 + 1

        wait_g(k0, 0)
        compute(0)

        @pl.when(k0 >= 1)
        def _():
            wait_id(k0, 0)

        sadd(k0, 0)

        @pl.when(k0 >= 1)
        def _():
            wait_sa(k0 - 1, 1)
            wait_is(k1, 1)

        issue_g(k1, 1)
        issue_id(k1, 1)

        @pl.when(k0 + 2 < NCH)
        def _():
            issue_is(k0 + 2, 0)

        wait_g(k1, 1)
        compute(1)
        wait_id(k1, 1)
        sadd(k1, 1)

        @pl.when(k1 + 1 < NCH)
        def _():
            wait_sa(k0, 0)
            wait_is(k1 + 1, 0)
            issue_g(k1 + 1, 0)
            issue_id(k1 + 1, 0)

            @pl.when(k1 + 2 < NCH)
            def _():
                issue_is(k1 + 2, 1)

        return carry

    lax.fori_loop(0, (NCH - 1) // 2, gstep, 0)
    wait_g(NCH - 1, 0)
    compute(0)
    wait_id(NCH - 1, 0)
    sadd(NCH - 1, 0)
    wait_sa(NCH - 2, 1)
    wait_sa(NCH - 1, 0)
    plsc.subcore_barrier()
    pltpu.sync_copy(shared.at[pl.ds(s * nstripe, nstripe)],
                    agg_hbm.at[c, pl.ds(s * nstripe, nstripe)])


_msg_agg = pl.kernel(
    _msg_agg_body,
    out_type=jax.ShapeDtypeStruct((NC, NP, D), jnp.float32),
    mesh=_mesh,
    compiler_params=_sc_params,
    scratch_types=[
        pltpu.VMEM((CH,), jnp.int32),
        pltpu.VMEM((CH,), jnp.int32),
        pltpu.VMEM((CH,), jnp.int32),
        pltpu.VMEM((CH,), jnp.int32),
        pltpu.VMEM((CH, D), jnp.float32),
        pltpu.VMEM((CH, D), jnp.float32),
        pltpu.VMEM((CH, D // 2), jnp.int32),
        pltpu.VMEM((CH, D // 2), jnp.int32),
        pltpu.VMEM_SHARED((NP, D), jnp.float32),
        pltpu.SemaphoreType.DMA,
        pltpu.SemaphoreType.DMA,
        pltpu.SemaphoreType.DMA,
        pltpu.SemaphoreType.DMA,
        pltpu.SemaphoreType.DMA,
        pltpu.SemaphoreType.DMA,
        pltpu.SemaphoreType.DMA,
        pltpu.SemaphoreType.DMA,
    ],
)


def _node_upd_body(agg_ref, xs_ref, wu_ref, bu_ref, wl_ref, bl_ref,
                   ls_ref, ld_ref):
    aggs = agg_ref[0] + agg_ref[1]
    x2 = jnp.maximum(
        jnp.dot(aggs, wu_ref[...], preferred_element_type=jnp.float32)
        + xs_ref[...] + bu_ref[...], 0.0)
    ls_ref[...] = jnp.dot(x2, wl_ref[0:D, :],
                          preferred_element_type=jnp.float32) + bl_ref[...]
    ld_ref[...] = jnp.dot(x2, wl_ref[2 * D:3 * D, :],
                          preferred_element_type=jnp.float32)


_node_upd = pl.pallas_call(
    _node_upd_body,
    out_shape=[
        jax.ShapeDtypeStruct((NP, D), jnp.float32),
        jax.ShapeDtypeStruct((NP, D), jnp.float32),
    ],
)


def _edge_out_body(ls_hbm, ld_hbm, wrl_hbm, src_hbm, dst_hbm,
                   cid_hbm, enc_hbm, t0_hbm, t2_hbm,
                   isa, ida, cid_v,
                   a0, a1, b0, b1, c0, c1, u0, u1, v0, v1,
                   sg0, sg1, so0, so1):
    wid = _wid()
    avs = (a0, a1)
    bvs = (b0, b1)
    cvs = (c0, c1)
    uvs = (u0, u1)
    vvs = (v0, v1)
    sgs = (sg0, sg1)
    sos = (so0, so1)

    pltpu.sync_copy(src_hbm.at[pl.ds(wid * EPT, EPT)], isa)
    pltpu.sync_copy(dst_hbm.at[pl.ds(wid * EPT, EPT)], ida)
    pltpu.sync_copy(cid_hbm, cid_v)

    def issue(k, b):
        sl = pl.ds(k * CH, CH)
        pltpu.async_copy(ls_hbm.at[isa.at[sl]], avs[b], sgs[b])
        pltpu.async_copy(ld_hbm.at[ida.at[sl]], bvs[b], sgs[b])
        pltpu.async_copy(wrl_hbm.at[pl.ds(wid * EPT + k * CH, CH)],
                         cvs[b], sgs[b])

    def wait_g(k, b):
        sl = pl.ds(k * CH, CH)
        pltpu.make_async_copy(ls_hbm.at[isa.at[sl]], avs[b], sgs[b]).wait()
        pltpu.make_async_copy(ld_hbm.at[ida.at[sl]], bvs[b], sgs[b]).wait()
        pltpu.make_async_copy(wrl_hbm.at[pl.ds(wid * EPT + k * CH, CH)],
                              cvs[b], sgs[b]).wait()

    def out(k, b):
        off = wid * EPT + k * CH
        pltpu.async_copy(avs[b], enc_hbm.at[pl.ds(off, CH)], sos[b])
        pltpu.async_copy(uvs[b], t0_hbm.at[pl.ds(off, CH)], sos[b])
        pltpu.async_copy(vvs[b], t2_hbm.at[pl.ds(off, CH)], sos[b])

    def wait_out(k, b):
        off = wid * EPT + k * CH
        pltpu.make_async_copy(avs[b], enc_hbm.at[pl.ds(off, CH)],
                              sos[b]).wait()
        pltpu.make_async_copy(uvs[b], t0_hbm.at[pl.ds(off, CH)],
                              sos[b]).wait()
        pltpu.make_async_copy(vvs[b], t2_hbm.at[pl.ds(off, CH)],
                              sos[b]).wait()

    MASKHI = jnp.full((16,), -65536, jnp.int32)

    def compute(k, b):
        av, bv, cv, uv, vv = avs[b], bvs[b], cvs[b], uvs[b], vvs[b]

        def rstep(ri, c2):
            for u in range(2):
                r = 2 * ri + u
                for g in range(4):
                    pk = cv[r, pl.ds(16 * g, 16)]
                    lo = plsc.bitcast(pk << 16, jnp.float32)
                    hi = plsc.bitcast(pk & MASKHI, jnp.float32)
                    sl = pl.ds(16 * g, 16)
                    sh = pl.ds(16 * g + D // 2, 16)
                    av[r, sl] = av[r, sl] + bv[r, sl] + lo
                    av[r, sh] = av[r, sh] + bv[r, sh] + hi
            return c2

        lax.fori_loop(0, CH // 2, rstep, 0)

        for i in range(CH // 16):
            sv = isa[pl.ds(k * CH + i * 16, 16)]
            dv = ida[pl.ds(k * CH + i * 16, 16)]
            uv[pl.ds(i * 16, 16)] = plsc.load_gather(cid_v, [sv])
            vv[pl.ds(i * 16, 16)] = plsc.load_gather(cid_v, [dv])

    issue(0, 0)

    def gstep(g, carry):
        k0 = 2 * g
        k1 = k0 + 1

        @pl.when(k0 >= 1)
        def _():
            wait_out(k0 - 1, 1)

        issue(k1, 1)
        wait_g(k0, 0)
        compute(k0, 0)
        out(k0, 0)

        @pl.when(k1 + 1 < NCH)
        def _():
            wait_out(k0, 0)
            issue(k1 + 1, 0)

        wait_g(k1, 1)
        compute(k1, 1)
        out(k1, 1)
        return carry

    lax.fori_loop(0, (NCH - 1) // 2, gstep, 0)
    wait_g(NCH - 1, 0)
    compute(NCH - 1, 0)
    out(NCH - 1, 0)
    wait_out(NCH - 2, 1)
    wait_out(NCH - 1, 0)


_edge_out = pl.kernel(
    _edge_out_body,
    out_type=[
        jax.ShapeDtypeStruct((NE, D), jnp.float32),
        jax.ShapeDtypeStruct((NE,), jnp.int32),
        jax.ShapeDtypeStruct((NE,), jnp.int32),
    ],
    mesh=_mesh,
    compiler_params=_sc_params,
    scratch_types=[
        pltpu.VMEM((EPT,), jnp.int32),
        pltpu.VMEM((EPT,), jnp.int32),
        pltpu.VMEM((NP,), jnp.int32),
        pltpu.VMEM((CH, D), jnp.float32),
        pltpu.VMEM((CH, D), jnp.float32),
        pltpu.VMEM((CH, D), jnp.float32),
        pltpu.VMEM((CH, D), jnp.float32),
        pltpu.VMEM((CH, D // 2), jnp.int32),
        pltpu.VMEM((CH, D // 2), jnp.int32),
        pltpu.VMEM((CH,), jnp.int32),
        pltpu.VMEM((CH,), jnp.int32),
        pltpu.VMEM((CH,), jnp.int32),
        pltpu.VMEM((CH,), jnp.int32),
        pltpu.SemaphoreType.DMA,
        pltpu.SemaphoreType.DMA,
        pltpu.SemaphoreType.DMA,
        pltpu.SemaphoreType.DMA,
    ],
)


def kernel(concept_ids, edge_index, edge_attr, concept_embedding,
           relation_embedding, W_msg, b_msg, W_self, W_upd, b_upd,
           W_lin, b_lin):
    src1 = edge_index[0]
    dst1 = edge_index[1]
    cid_pad = jnp.concatenate(
        [concept_ids, jnp.zeros((NP - NN,), jnp.int32)])
    rp = jnp.pad(relation_embedding, ((0, NRELP - NREL), (0, 0)))

    x = _gather_x(concept_embedding, cid_pad)
    xm, xs, relcat = _node_pre(x, W_msg, b_msg, W_self, rp, W_lin)
    wrm = _edge_bias(edge_attr, relcat[:, 0:D])
    wrl = _edge_bias(edge_attr, relcat[:, D:2 * D])
    agg2 = _msg_agg(xm, wrm, src1, dst1)
    ls, ld = _node_upd(agg2, xs, W_upd, b_upd, W_lin, b_lin)
    enc, t0, t2 = _edge_out(ls, ld, wrl, src1, dst1, cid_pad)
    tid = jnp.stack(
        [t0, edge_attr[:, 0].astype(jnp.int32), t2], axis=1)
    return enc, tid


# speedup vs baseline: 2.3442x; 1.0643x over previous
import jax
import jax.numpy as jnp
from jax import lax
from jax.experimental import pallas as pl
from jax.experimental.pallas import tpu as pltpu
from jax.experimental.pallas import tpu_sc as plsc

D = 128
NN = 10000
NE = 320000
NREL = 38
NRELP = 40
NP = 10240
NC, NS = 2, 16
NW = NC * NS
CH = 80
EPT = NE // NW
NCH = EPT // CH
APT = NP // NW

_mesh = plsc.VectorSubcoreMesh(
    core_axis_name="c", subcore_axis_name="s", num_cores=NC, num_subcores=NS)
_sc_params = pltpu.CompilerParams(needs_layout_passes=False)


def _wid():
    return lax.axis_index("s") * NC + lax.axis_index("c")


def _gather_x_body(ce_hbm, cid_hbm, x_hbm, idx_v, rows_v, sem):
    base = _wid() * APT

    def step(k, carry):
        off = base + k * CH
        pltpu.sync_copy(cid_hbm.at[pl.ds(off, CH)], idx_v)
        pltpu.async_copy(ce_hbm.at[idx_v], rows_v, sem).wait()
        pltpu.sync_copy(rows_v, x_hbm.at[pl.ds(off, CH)])
        return carry

    lax.fori_loop(0, APT // CH, step, 0)


_gather_x = pl.kernel(
    _gather_x_body,
    out_type=jax.ShapeDtypeStruct((NP, D), jnp.float32),
    mesh=_mesh,
    compiler_params=_sc_params,
    scratch_types=[
        pltpu.VMEM((CH,), jnp.int32),
        pltpu.VMEM((CH, D), jnp.float32),
        pltpu.SemaphoreType.DMA,
    ],
)


def _node_pre_body(x_ref, wm_ref, bm_ref, ws_ref, rp_ref, wl_ref,
                   xm_ref, xs_ref, relcat_ref):
    x = x_ref[...]
    xm_ref[...] = jnp.dot(x, wm_ref[0:D, :],
                          preferred_element_type=jnp.float32) + bm_ref[...]
    xs_ref[...] = jnp.dot(x, ws_ref[...], preferred_element_type=jnp.float32)
    rp = rp_ref[...]
    rm = jnp.dot(rp, wm_ref[D:2 * D, :], preferred_element_type=jnp.float32)
    rl = jnp.dot(rp, wl_ref[D:2 * D, :], preferred_element_type=jnp.float32)
    relcat_ref[...] = jnp.concatenate([rm, rl], axis=1)


_node_pre = pl.pallas_call(
    _node_pre_body,
    out_shape=[
        jax.ShapeDtypeStruct((NP, D), jnp.float32),
        jax.ShapeDtypeStruct((NP, D), jnp.float32),
        jax.ShapeDtypeStruct((NRELP, 2 * D), jnp.float32),
    ],
)


EB = 2000


def _pack_bf16(lo, hi):
    rl = lo.astype(jnp.bfloat16).astype(jnp.float32)
    rh = hi.astype(jnp.bfloat16).astype(jnp.float32)
    il = lax.shift_right_logical(
        lax.bitcast_convert_type(rl, jnp.int32), 16)
    ih = lax.bitcast_convert_type(rh, jnp.int32)
    return il | ih


def _edge_bias_body(attr_ref, tab_ref, out_ref):
    attr = attr_ref[...]
    reli = attr[:, 0:1].astype(jnp.int32)
    w = attr[:, 1:2]
    io = lax.broadcasted_iota(jnp.int32, (EB, NRELP), 1)
    ohw = jnp.where(reli == io, w, 0.0)
    big = jnp.dot(ohw, tab_ref[...], preferred_element_type=jnp.float32)
    out_ref[...] = _pack_bf16(big[:, 0:D // 2], big[:, D // 2:D])


_edge_bias = pl.pallas_call(
    _edge_bias_body,
    grid=(NE // EB,),
    in_specs=[
        pl.BlockSpec((EB, 2), lambda i: (i, 0)),
        pl.BlockSpec((NRELP, D), lambda i: (0, 0)),
    ],
    out_specs=pl.BlockSpec((EB, D // 2), lambda i: (i, 0)),
    out_shape=jax.ShapeDtypeStruct((NE, D // 2), jnp.int32),
)


def _msg_agg_body(xm_hbm, wrm_hbm, src_hbm, dst_hbm, agg_hbm,
                  is0, is1, id0, id1, r0, r1, w0, w1, shared,
                  sg0, sg1, sa0, sa1, si0, si1, sd0, sd1):
    c = lax.axis_index("c")
    s = lax.axis_index("s")
    wid = s * NC + c
    nstripe = NP // NS
    iss = (is0, is1)
    ids = (id0, id1)
    rows = (r0, r1)
    wrms = (w0, w1)
    sgs = (sg0, sg1)
    sas = (sa0, sa1)
    sis = (si0, si1)
    sds = (sd0, sd1)

    zero = jnp.zeros((16,), jnp.float32)
    for r in range(40):
        for j in range(8):
            r0[r, pl.ds(16 * j, 16)] = zero

    def zstep(i, carry):
        pltpu.sync_copy(r0.at[pl.ds(0, 40)],
                        shared.at[pl.ds(s * nstripe + i * 40, 40)])
        return carry

    lax.fori_loop(0, nstripe // 40, zstep, 0)
    plsc.subcore_barrier()

    def issue_is(k, b):
        pltpu.async_copy(src_hbm.at[pl.ds(wid * EPT + k * CH, CH)],
                         iss[b], sis[b])

    def wait_is(k, b):
        pltpu.make_async_copy(src_hbm.at[pl.ds(wid * EPT + k * CH, CH)],
                              iss[b], sis[b]).wait()

    def issue_id(k, b):
        pltpu.async_copy(dst_hbm.at[pl.ds(wid * EPT + k * CH, CH)],
                         ids[b], sds[b])

    def wait_id(k, b):
        pltpu.make_async_copy(dst_hbm.at[pl.ds(wid * EPT + k * CH, CH)],
                              ids[b], sds[b]).wait()

    MASKHI = jnp.full((16,), -65536, jnp.int32)

    def issue_g(k, b):
        pltpu.async_copy(xm_hbm.at[iss[b]], rows[b], sgs[b])
        pltpu.async_copy(wrm_hbm.at[pl.ds(wid * EPT + k * CH, CH)],
                         wrms[b], sgs[b])

    def wait_g(k, b):
        pltpu.make_async_copy(xm_hbm.at[iss[b]], rows[b], sgs[b]).wait()
        pltpu.make_async_copy(wrm_hbm.at[pl.ds(wid * EPT + k * CH, CH)],
                              wrms[b], sgs[b]).wait()

    def sadd(k, b):
        pltpu.async_copy(rows[b], shared.at[ids[b]], sas[b], add=True)

    def wait_sa(k, b):
        pltpu.make_async_copy(rows[b], shared.at[ids[b]], sas[b]).wait()

    def compute(b):
        rv, wv = rows[b], wrms[b]

        def rstep(ri, c2):
            for u in range(4):
                r = 4 * ri + u
                for g in range(4):
                    pk = wv[r, pl.ds(16 * g, 16)]
                    lo = plsc.bitcast(pk << 16, jnp.float32)
                    hi = plsc.bitcast(pk & MASKHI, jnp.float32)
                    sl = pl.ds(16 * g, 16)
                    sh = pl.ds(16 * g + D // 2, 16)
                    rv[r, sl] = jnp.maximum(rv[r, sl] + lo, 0.0)
                    rv[r, sh] = jnp.maximum(rv[r, sh] + hi, 0.0)
            return c2

        lax.fori_loop(0, CH // 4, rstep, 0)

    pltpu.sync_copy(src_hbm.at[pl.ds(wid * EPT, CH)], is0)
    pltpu.sync_copy(dst_hbm.at[pl.ds(wid * EPT, CH)], id0)
    pltpu.sync_copy(src_hbm.at[pl.ds(wid * EPT + CH, CH)], is1)
    issue_g(0, 0)

    def gstep(g, carry):
        k0 = 2 * g
        k1 = k0 + 1

        wait_g(k0, 0)

        @pl.when(k0 >= 1)
        def _():
            wait_sa(k0 - 1, 1)
            wait_is(k1, 1)

        issue_g(k1, 1)
        issue_id(k1, 1)

        @pl.when(k0 + 2 < NCH)
        def _():
            issue_is(k0 + 2, 0)

        compute(0)

        @pl.when(k0 >= 1)
        def _():
            wait_id(k0, 0)

        sadd(k0, 0)

        wait_g(k1, 1)

        @pl.when(k1 + 1 < NCH)
        def _():
            wait_sa(k0, 0)
            wait_is(k1 + 1, 0)
            issue_g(k1 + 1, 0)
            issue_id(k1 + 1, 0)

            @pl.when(k1 + 2 < NCH)
            def _():
                issue_is(k1 + 2, 1)

        compute(1)
        wait_id(k1, 1)
        sadd(k1, 1)
        return carry

    lax.fori_loop(0, (NCH - 1) // 2, gstep, 0)
    wait_g(NCH - 1, 0)
    compute(0)
    wait_id(NCH - 1, 0)
    sadd(NCH - 1, 0)
    wait_sa(NCH - 2, 1)
    wait_sa(NCH - 1, 0)
    plsc.subcore_barrier()
    pltpu.sync_copy(shared.at[pl.ds(s * nstripe, nstripe)],
                    agg_hbm.at[c, pl.ds(s * nstripe, nstripe)])


_msg_agg = pl.kernel(
    _msg_agg_body,
    out_type=jax.ShapeDtypeStruct((NC, NP, D), jnp.float32),
    mesh=_mesh,
    compiler_params=_sc_params,
    scratch_types=[
        pltpu.VMEM((CH,), jnp.int32),
        pltpu.VMEM((CH,), jnp.int32),
        pltpu.VMEM((CH,), jnp.int32),
        pltpu.VMEM((CH,), jnp.int32),
        pltpu.VMEM((CH, D), jnp.float32),
        pltpu.VMEM((CH, D), jnp.float32),
        pltpu.VMEM((CH, D // 2), jnp.int32),
        pltpu.VMEM((CH, D // 2), jnp.int32),
        pltpu.VMEM_SHARED((NP, D), jnp.float32),
        pltpu.SemaphoreType.DMA,
        pltpu.SemaphoreType.DMA,
        pltpu.SemaphoreType.DMA,
        pltpu.SemaphoreType.DMA,
        pltpu.SemaphoreType.DMA,
        pltpu.SemaphoreType.DMA,
        pltpu.SemaphoreType.DMA,
        pltpu.SemaphoreType.DMA,
    ],
)


def _node_upd_body(agg_ref, xs_ref, wu_ref, bu_ref, wl_ref, bl_ref,
                   ls_ref, ld_ref):
    aggs = agg_ref[0] + agg_ref[1]
    x2 = jnp.maximum(
        jnp.dot(aggs, wu_ref[...], preferred_element_type=jnp.float32)
        + xs_ref[...] + bu_ref[...], 0.0)
    ls_ref[...] = jnp.dot(x2, wl_ref[0:D, :],
                          preferred_element_type=jnp.float32) + bl_ref[...]
    ld_ref[...] = jnp.dot(x2, wl_ref[2 * D:3 * D, :],
                          preferred_element_type=jnp.float32)


_node_upd = pl.pallas_call(
    _node_upd_body,
    out_shape=[
        jax.ShapeDtypeStruct((NP, D), jnp.float32),
        jax.ShapeDtypeStruct((NP, D), jnp.float32),
    ],
)


def _edge_out_body(ls_hbm, ld_hbm, wrl_hbm, src_hbm, dst_hbm,
                   cid_hbm, enc_hbm, t0_hbm, t2_hbm,
                   isa, ida, cid_v,
                   a0, a1, b0, b1, c0, c1, u0, u1, v0, v1,
                   sg0, sg1, so0, so1):
    wid = _wid()
    avs = (a0, a1)
    bvs = (b0, b1)
    cvs = (c0, c1)
    uvs = (u0, u1)
    vvs = (v0, v1)
    sgs = (sg0, sg1)
    sos = (so0, so1)

    pltpu.sync_copy(src_hbm.at[pl.ds(wid * EPT, EPT)], isa)
    pltpu.sync_copy(dst_hbm.at[pl.ds(wid * EPT, EPT)], ida)
    pltpu.sync_copy(cid_hbm, cid_v)

    def issue(k, b):
        sl = pl.ds(k * CH, CH)
        pltpu.async_copy(ls_hbm.at[isa.at[sl]], avs[b], sgs[b])
        pltpu.async_copy(ld_hbm.at[ida.at[sl]], bvs[b], sgs[b])
        pltpu.async_copy(wrl_hbm.at[pl.ds(wid * EPT + k * CH, CH)],
                         cvs[b], sgs[b])

    def wait_g(k, b):
        sl = pl.ds(k * CH, CH)
        pltpu.make_async_copy(ls_hbm.at[isa.at[sl]], avs[b], sgs[b]).wait()
        pltpu.make_async_copy(ld_hbm.at[ida.at[sl]], bvs[b], sgs[b]).wait()
        pltpu.make_async_copy(wrl_hbm.at[pl.ds(wid * EPT + k * CH, CH)],
                              cvs[b], sgs[b]).wait()

    def out(k, b):
        off = wid * EPT + k * CH
        pltpu.async_copy(avs[b], enc_hbm.at[pl.ds(off, CH)], sos[b])
        pltpu.async_copy(uvs[b], t0_hbm.at[pl.ds(off, CH)], sos[b])
        pltpu.async_copy(vvs[b], t2_hbm.at[pl.ds(off, CH)], sos[b])

    def wait_out(k, b):
        off = wid * EPT + k * CH
        pltpu.make_async_copy(avs[b], enc_hbm.at[pl.ds(off, CH)],
                              sos[b]).wait()
        pltpu.make_async_copy(uvs[b], t0_hbm.at[pl.ds(off, CH)],
                              sos[b]).wait()
        pltpu.make_async_copy(vvs[b], t2_hbm.at[pl.ds(off, CH)],
                              sos[b]).wait()

    MASKHI = jnp.full((16,), -65536, jnp.int32)

    def compute(k, b):
        av, bv, cv, uv, vv = avs[b], bvs[b], cvs[b], uvs[b], vvs[b]

        def rstep(ri, c2):
            for u in range(2):
                r = 2 * ri + u
                for g in range(4):
                    pk = cv[r, pl.ds(16 * g, 16)]
                    lo = plsc.bitcast(pk << 16, jnp.float32)
                    hi = plsc.bitcast(pk & MASKHI, jnp.float32)
                    sl = pl.ds(16 * g, 16)
                    sh = pl.ds(16 * g + D // 2, 16)
                    av[r, sl] = av[r, sl] + bv[r, sl] + lo
                    av[r, sh] = av[r, sh] + bv[r, sh] + hi
            return c2

        lax.fori_loop(0, CH // 2, rstep, 0)

        for i in range(CH // 16):
            sv = isa[pl.ds(k * CH + i * 16, 16)]
            dv = ida[pl.ds(k * CH + i * 16, 16)]
            uv[pl.ds(i * 16, 16)] = plsc.load_gather(cid_v, [sv])
            vv[pl.ds(i * 16, 16)] = plsc.load_gather(cid_v, [dv])

    issue(0, 0)

    def gstep(g, carry):
        k0 = 2 * g
        k1 = k0 + 1

        @pl.when(k0 >= 1)
        def _():
            wait_out(k0 - 1, 1)

        issue(k1, 1)
        wait_g(k0, 0)
        compute(k0, 0)
        out(k0, 0)

        @pl.when(k1 + 1 < NCH)
        def _():
            wait_out(k0, 0)
            issue(k1 + 1, 0)

        wait_g(k1, 1)
        compute(k1, 1)
        out(k1, 1)
        return carry

    lax.fori_loop(0, (NCH - 1) // 2, gstep, 0)
    wait_g(NCH - 1, 0)
    compute(NCH - 1, 0)
    out(NCH - 1, 0)
    wait_out(NCH - 2, 1)
    wait_out(NCH - 1, 0)


_edge_out = pl.kernel(
    _edge_out_body,
    out_type=[
        jax.ShapeDtypeStruct((NE, D), jnp.float32),
        jax.ShapeDtypeStruct((NE,), jnp.int32),
        jax.ShapeDtypeStruct((NE,), jnp.int32),
    ],
    mesh=_mesh,
    compiler_params=_sc_params,
    scratch_types=[
        pltpu.VMEM((EPT,), jnp.int32),
        pltpu.VMEM((EPT,), jnp.int32),
        pltpu.VMEM((NP,), jnp.int32),
        pltpu.VMEM((CH, D), jnp.float32),
        pltpu.VMEM((CH, D), jnp.float32),
        pltpu.VMEM((CH, D), jnp.float32),
        pltpu.VMEM((CH, D), jnp.float32),
        pltpu.VMEM((CH, D // 2), jnp.int32),
        pltpu.VMEM((CH, D // 2), jnp.int32),
        pltpu.VMEM((CH,), jnp.int32),
        pltpu.VMEM((CH,), jnp.int32),
        pltpu.VMEM((CH,), jnp.int32),
        pltpu.VMEM((CH,), jnp.int32),
        pltpu.SemaphoreType.DMA,
        pltpu.SemaphoreType.DMA,
        pltpu.SemaphoreType.DMA,
        pltpu.SemaphoreType.DMA,
    ],
)


def kernel(concept_ids, edge_index, edge_attr, concept_embedding,
           relation_embedding, W_msg, b_msg, W_self, W_upd, b_upd,
           W_lin, b_lin):
    src1 = edge_index[0]
    dst1 = edge_index[1]
    cid_pad = jnp.concatenate(
        [concept_ids, jnp.zeros((NP - NN,), jnp.int32)])
    rp = jnp.pad(relation_embedding, ((0, NRELP - NREL), (0, 0)))

    x = _gather_x(concept_embedding, cid_pad)
    xm, xs, relcat = _node_pre(x, W_msg, b_msg, W_self, rp, W_lin)
    wrm = _edge_bias(edge_attr, relcat[:, 0:D])
    wrl = _edge_bias(edge_attr, relcat[:, D:2 * D])
    agg2 = _msg_agg(xm, wrm, src1, dst1)
    ls, ld = _node_upd(agg2, xs, W_upd, b_upd, W_lin, b_lin)
    enc, t0, t2 = _edge_out(ls, ld, wrl, src1, dst1, cid_pad)
    tid = jnp.stack(
        [t0, edge_attr[:, 0].astype(jnp.int32), t2], axis=1)
    return enc, tid
